# Initial kernel scaffold; baseline (speedup 1.0000x reference)
import functools

import jax
import jax.numpy as jnp
from jax import lax
from jax.experimental import pallas as pl
from jax.experimental.pallas import tpu as pltpu
from jax.experimental.pallas import tpu_sc as plsc

N = 10000
E = 320000
DH = 16
CH = 128
NCH = E // CH
NW = 32
FULL = NCH // NW
REM = NCH - FULL * NW
RPW = N // 16

_mesh = plsc.VectorSubcoreMesh(core_axis_name="c", subcore_axis_name="s")


@functools.partial(
    pl.kernel,
    out_type=jax.ShapeDtypeStruct((2, N, DH), jnp.float32),
    mesh=_mesh,
    scratch_types=[
        pltpu.VMEM((CH,), jnp.int32),
        pltpu.VMEM((CH,), jnp.int32),
        pltpu.VMEM((CH, DH), jnp.float32),
        pltpu.VMEM_SHARED((N, DH), jnp.float32),
    ],
)
def _sc_segsum(y_hbm, src_hbm, dst_hbm, zeros_hbm, out_hbm,
               idx_s, idx_d, rows, acc):
    c = lax.axis_index("c")
    s = lax.axis_index("s")
    w = c * 16 + s

    base_r = s * RPW
    pltpu.sync_copy(zeros_hbm.at[pl.ds(base_r, RPW)], acc.at[pl.ds(base_r, RPW)])
    plsc.subcore_barrier()

    def do_chunk(chunk_id):
        base_e = chunk_id * CH
        pltpu.sync_copy(src_hbm.at[pl.ds(base_e, CH)], idx_s)
        pltpu.sync_copy(dst_hbm.at[pl.ds(base_e, CH)], idx_d)
        pltpu.sync_copy(y_hbm.at[idx_s], rows)
        pltpu.sync_copy(rows, acc.at[idx_d], add=True)

    def body(j, carry):
        do_chunk(w * FULL + j)
        return carry

    lax.fori_loop(0, FULL, body, 0)

    @pl.when(w < REM)
    def _():
        do_chunk(NW * FULL + w)

    plsc.subcore_barrier()
    pltpu.sync_copy(acc.at[pl.ds(base_r, RPW)], out_hbm.at[c, pl.ds(base_r, RPW)])


@functools.partial(
    pl.kernel,
    out_type=jax.ShapeDtypeStruct((2, N, DH), jnp.float32),
    mesh=_mesh,
    scratch_types=[
        pltpu.VMEM((CH,), jnp.int32),
        pltpu.VMEM((CH, DH), jnp.float32),
        pltpu.VMEM_SHARED((N, DH), jnp.float32),
    ],
)
def _sc_degree(dst_hbm, zeros_hbm, ones_hbm, out_hbm, idx_d, ones_v, acc):
    c = lax.axis_index("c")
    s = lax.axis_index("s")
    w = c * 16 + s

    base_r = s * RPW
    pltpu.sync_copy(zeros_hbm.at[pl.ds(base_r, RPW)], acc.at[pl.ds(base_r, RPW)])
    pltpu.sync_copy(ones_hbm, ones_v)
    plsc.subcore_barrier()

    def do_chunk(chunk_id):
        base_e = chunk_id * CH
        pltpu.sync_copy(dst_hbm.at[pl.ds(base_e, CH)], idx_d)
        pltpu.sync_copy(ones_v, acc.at[idx_d], add=True)

    def body(j, carry):
        do_chunk(w * FULL + j)
        return carry

    lax.fori_loop(0, FULL, body, 0)

    @pl.when(w < REM)
    def _():
        do_chunk(NW * FULL + w)

    plsc.subcore_barrier()
    pltpu.sync_copy(acc.at[pl.ds(base_r, RPW)], out_hbm.at[c, pl.ds(base_r, RPW)])


BN = 1000


def _dinv_of(dp_ref):
    cnt = dp_ref[0, :, 0:1] + dp_ref[1, :, 0:1]
    return lax.rsqrt(cnt + 1.0)


def _mm_body(x_ref, w_ref, o_ref):
    o_ref[...] = lax.dot_general(
        x_ref[...], w_ref[...], (((1,), (0,)), ((), ())),
        precision=lax.Precision.HIGHEST, preferred_element_type=jnp.float32)


def _y1_body(dp_ref, xw_ref, y_ref):
    y_ref[...] = xw_ref[...] * _dinv_of(dp_ref)


def _layer2_body(mp_ref, dp_ref, xw_ref, b1_ref, w2_ref, y2_ref, xw2_ref):
    dinv = _dinv_of(dp_ref)
    agg = mp_ref[0] + mp_ref[1]
    h = dinv * agg + (dinv * dinv) * xw_ref[...] + b1_ref[...]
    h = jnp.maximum(h, 0.0)
    xw2 = lax.dot_general(h, w2_ref[...], (((1,), (0,)), ((), ())),
                          precision=lax.Precision.HIGHEST,
                          preferred_element_type=jnp.float32)
    xw2_ref[...] = xw2
    y2_ref[...] = xw2 * dinv


def _out_body(mp_ref, dp_ref, xw2_ref, b2_ref, o_ref):
    dinv = _dinv_of(dp_ref)
    agg = mp_ref[0] + mp_ref[1]
    o = dinv * agg + (dinv * dinv) * xw2_ref[...] + b2_ref[...]
    m = jnp.max(o, axis=1, keepdims=True)
    lse = m + jnp.log(jnp.sum(jnp.exp(o - m), axis=1, keepdims=True))
    o_ref[...] = (o - lse)[:, 0:2]


def _rows_spec(width):
    return pl.BlockSpec((BN, width), lambda i: (i, 0))


def _pair_spec():
    return pl.BlockSpec((2, BN, DH), lambda i: (0, i, 0))


def _full_spec(shape):
    return pl.BlockSpec(shape, lambda i: tuple(0 for _ in shape))


def kernel(x, edge_index, W1, b1, W2, b2):
    f32 = jnp.float32
    src = edge_index[0]
    dst = edge_index[1]
    zeros_nk = jnp.zeros((N, DH), f32)
    ones_blk = jnp.ones((CH, DH), f32)
    W2p = jnp.zeros((DH, DH), f32).at[:, :2].set(W2)
    b1r = b1.reshape(1, DH)
    b2p = jnp.full((1, DH), -1e30, f32).at[0, :2].set(b2)

    grid = (N // BN,)

    dp = _sc_degree(dst, zeros_nk, ones_blk)

    xw1 = pl.pallas_call(
        _mm_body, grid=grid,
        in_specs=[_rows_spec(128), _full_spec((128, DH))],
        out_specs=_rows_spec(DH),
        out_shape=jax.ShapeDtypeStruct((N, DH), f32),
    )(x, W1)

    y1 = pl.pallas_call(
        _y1_body, grid=grid,
        in_specs=[_pair_spec(), _rows_spec(DH)],
        out_specs=_rows_spec(DH),
        out_shape=jax.ShapeDtypeStruct((N, DH), f32),
    )(dp, xw1)

    mp1 = _sc_segsum(y1, src, dst, zeros_nk)

    y2, xw2 = pl.pallas_call(
        _layer2_body, grid=grid,
        in_specs=[_pair_spec(), _pair_spec(), _rows_spec(DH),
                  _full_spec((1, DH)), _full_spec((DH, DH))],
        out_specs=[_rows_spec(DH), _rows_spec(DH)],
        out_shape=[jax.ShapeDtypeStruct((N, DH), f32),
                   jax.ShapeDtypeStruct((N, DH), f32)],
    )(mp1, dp, xw1, b1r, W2p)

    mp2 = _sc_segsum(y2, src, dst, zeros_nk)

    out = pl.pallas_call(
        _out_body, grid=grid,
        in_specs=[_pair_spec(), _pair_spec(), _rows_spec(DH),
                  _full_spec((1, DH))],
        out_specs=_rows_spec(2),
        out_shape=jax.ShapeDtypeStruct((N, 2), f32),
    )(mp2, dp, xw2, b2p)

    return out


# speedup vs baseline: 20.8272x; 20.8272x over previous
import functools

import jax
import jax.numpy as jnp
from jax import lax
from jax.experimental import pallas as pl
from jax.experimental.pallas import tpu as pltpu
from jax.experimental.pallas import tpu_sc as plsc

N = 10000
E = 320000
DH = 16
CH = 128
NCH = E // CH
NW = 32
FULL = NCH // NW
REM = NCH - FULL * NW
RPW = 624
TAIL = N - 16 * RPW


def _sc_segsum_body(y_hbm, src_hbm, dst_hbm, zeros_hbm, out_hbm,
                    idx_s, idx_d, rows, acc):
    c = lax.axis_index("c")
    s = lax.axis_index("s")
    w = c * 16 + s

    base_r = s * RPW
    pltpu.sync_copy(zeros_hbm.at[pl.ds(base_r, RPW)], acc.at[pl.ds(base_r, RPW)])

    @pl.when(s == 0)
    def _():
        pltpu.sync_copy(zeros_hbm.at[pl.ds(16 * RPW, TAIL)],
                        acc.at[pl.ds(16 * RPW, TAIL)])

    plsc.subcore_barrier()

    def do_chunk(chunk_id):
        base_e = chunk_id * CH
        pltpu.sync_copy(src_hbm.at[pl.ds(base_e, CH)], idx_s)
        pltpu.sync_copy(dst_hbm.at[pl.ds(base_e, CH)], idx_d)
        pltpu.sync_copy(y_hbm.at[idx_s], rows)
        pltpu.sync_copy(rows, acc.at[idx_d], add=True)

    def body(j, carry):
        do_chunk(w * FULL + j)
        return carry

    lax.fori_loop(0, FULL, body, 0)

    @pl.when(w < REM)
    def _():
        do_chunk(NW * FULL + w)

    plsc.subcore_barrier()
    pltpu.sync_copy(acc.at[pl.ds(base_r, RPW)], out_hbm.at[c, pl.ds(base_r, RPW)])

    @pl.when(s == 0)
    def _():
        pltpu.sync_copy(acc.at[pl.ds(16 * RPW, TAIL)],
                        out_hbm.at[c, pl.ds(16 * RPW, TAIL)])


def _sc_degree_body(dst_hbm, zeros_hbm, ones_hbm, out_hbm, idx_d, ones_v, acc):
    c = lax.axis_index("c")
    s = lax.axis_index("s")
    w = c * 16 + s

    base_r = s * RPW
    pltpu.sync_copy(zeros_hbm.at[pl.ds(base_r, RPW)], acc.at[pl.ds(base_r, RPW)])

    @pl.when(s == 0)
    def _():
        pltpu.sync_copy(zeros_hbm.at[pl.ds(16 * RPW, TAIL)],
                        acc.at[pl.ds(16 * RPW, TAIL)])

    pltpu.sync_copy(ones_hbm, ones_v)
    plsc.subcore_barrier()

    def do_chunk(chunk_id):
        base_e = chunk_id * CH
        pltpu.sync_copy(dst_hbm.at[pl.ds(base_e, CH)], idx_d)
        pltpu.sync_copy(ones_v, acc.at[idx_d], add=True)

    def body(j, carry):
        do_chunk(w * FULL + j)
        return carry

    lax.fori_loop(0, FULL, body, 0)

    @pl.when(w < REM)
    def _():
        do_chunk(NW * FULL + w)

    plsc.subcore_barrier()
    pltpu.sync_copy(acc.at[pl.ds(base_r, RPW)], out_hbm.at[c, pl.ds(base_r, RPW)])

    @pl.when(s == 0)
    def _():
        pltpu.sync_copy(acc.at[pl.ds(16 * RPW, TAIL)],
                        out_hbm.at[c, pl.ds(16 * RPW, TAIL)])


@functools.lru_cache(maxsize=None)
def _sc_kernels():
    mesh = plsc.VectorSubcoreMesh(core_axis_name="c", subcore_axis_name="s",
                                  num_cores=2, num_subcores=16)
    out_t = jax.ShapeDtypeStruct((2, N, DH), jnp.float32)
    cp = pltpu.CompilerParams(use_tc_tiling_on_sc=False)
    segsum = pl.kernel(
        _sc_segsum_body, out_type=out_t, mesh=mesh, compiler_params=cp,
        scratch_types=[
            pltpu.VMEM((CH,), jnp.int32),
            pltpu.VMEM((CH,), jnp.int32),
            pltpu.VMEM((CH, DH), jnp.float32),
            pltpu.VMEM_SHARED((N, DH), jnp.float32),
        ])
    degree = pl.kernel(
        _sc_degree_body, out_type=out_t, mesh=mesh, compiler_params=cp,
        scratch_types=[
            pltpu.VMEM((CH,), jnp.int32),
            pltpu.VMEM((CH, DH), jnp.float32),
            pltpu.VMEM_SHARED((N, DH), jnp.float32),
        ])
    return segsum, degree


BN = 1000


def _dinv_of(dp_ref):
    cnt = dp_ref[0, :, 0:1] + dp_ref[1, :, 0:1]
    return lax.rsqrt(cnt + 1.0)


def _mm_body(x_ref, w_ref, o_ref):
    o_ref[...] = lax.dot_general(
        x_ref[...], w_ref[...], (((1,), (0,)), ((), ())),
        precision=lax.Precision.HIGHEST, preferred_element_type=jnp.float32)


def _y1_body(dp_ref, xw_ref, y_ref):
    y_ref[...] = xw_ref[...] * _dinv_of(dp_ref)


def _layer2_body(mp_ref, dp_ref, xw_ref, b1_ref, w2_ref, y2_ref, xw2_ref):
    dinv = _dinv_of(dp_ref)
    agg = mp_ref[0] + mp_ref[1]
    h = dinv * agg + (dinv * dinv) * xw_ref[...] + b1_ref[...]
    h = jnp.maximum(h, 0.0)
    xw2 = lax.dot_general(h, w2_ref[...], (((1,), (0,)), ((), ())),
                          precision=lax.Precision.HIGHEST,
                          preferred_element_type=jnp.float32)
    xw2_ref[...] = xw2
    y2_ref[...] = xw2 * dinv


def _out_body(mp_ref, dp_ref, xw2_ref, b2_ref, o_ref):
    dinv = _dinv_of(dp_ref)
    agg = mp_ref[0] + mp_ref[1]
    o = dinv * agg + (dinv * dinv) * xw2_ref[...] + b2_ref[...]
    m = jnp.max(o, axis=1, keepdims=True)
    lse = m + jnp.log(jnp.sum(jnp.exp(o - m), axis=1, keepdims=True))
    o_ref[...] = (o - lse)[:, 0:2]


def _rows_spec(width):
    return pl.BlockSpec((BN, width), lambda i: (i, 0))


def _pair_spec():
    return pl.BlockSpec((2, BN, DH), lambda i: (0, i, 0))


def _full_spec(shape):
    return pl.BlockSpec(shape, lambda i: tuple(0 for _ in shape))


def kernel(x, edge_index, W1, b1, W2, b2):
    f32 = jnp.float32
    src = edge_index[0]
    dst = edge_index[1]
    zeros_nk = jnp.zeros((N, DH), f32)
    ones_blk = jnp.ones((CH, DH), f32)
    W2p = jnp.zeros((DH, DH), f32).at[:, :2].set(W2)
    b1r = b1.reshape(1, DH)
    b2p = jnp.full((1, DH), -1e30, f32).at[0, :2].set(b2)

    grid = (N // BN,)
    sc_segsum, sc_degree = _sc_kernels()

    dp = sc_degree(dst, zeros_nk, ones_blk)

    xw1 = pl.pallas_call(
        _mm_body, grid=grid,
        in_specs=[_rows_spec(128), _full_spec((128, DH))],
        out_specs=_rows_spec(DH),
        out_shape=jax.ShapeDtypeStruct((N, DH), f32),
    )(x, W1)

    y1 = pl.pallas_call(
        _y1_body, grid=grid,
        in_specs=[_pair_spec(), _rows_spec(DH)],
        out_specs=_rows_spec(DH),
        out_shape=jax.ShapeDtypeStruct((N, DH), f32),
    )(dp, xw1)

    mp1 = sc_segsum(y1, src, dst, zeros_nk)

    y2, xw2 = pl.pallas_call(
        _layer2_body, grid=grid,
        in_specs=[_pair_spec(), _pair_spec(), _rows_spec(DH),
                  _full_spec((1, DH)), _full_spec((DH, DH))],
        out_specs=[_rows_spec(DH), _rows_spec(DH)],
        out_shape=[jax.ShapeDtypeStruct((N, DH), f32),
                   jax.ShapeDtypeStruct((N, DH), f32)],
    )(mp1, dp, xw1, b1r, W2p)

    mp2 = sc_segsum(y2, src, dst, zeros_nk)

    out = pl.pallas_call(
        _out_body, grid=grid,
        in_specs=[_pair_spec(), _pair_spec(), _rows_spec(DH),
                  _full_spec((1, DH))],
        out_specs=_rows_spec(2),
        out_shape=jax.ShapeDtypeStruct((N, 2), f32),
    )(mp2, dp, xw2, b2p)

    return out
